# Initial kernel scaffold; baseline (speedup 1.0000x reference)
import jax
import jax.numpy as jnp
from jax.experimental import pallas as pl


def kernel(X, tables, W_lin, b_lin, W1, b1, W2, b2, W_out):
    raise NotImplementedError("write your pallas kernel here")


# speedup vs baseline: 1.9377x; 1.9377x over previous
import functools

import jax
import jax.numpy as jnp
from jax import lax
from jax.experimental import pallas as pl
from jax.experimental.pallas import tpu as pltpu
from jax.experimental.pallas import tpu_sc as plsc

B = 16384
N_SPARSE = 26
N_DENSE = 13
VOCAB = 100000
EDIM = 16
HID1 = 256
HID2 = 128
N_FEAT = N_SPARSE + N_DENSE
SPARSE_DIM = N_SPARSE * EDIM

NC = 2
NSUB = 16
NW = NC * NSUB
ROWS_W = B // NW
IDX_W = ROWS_W * N_SPARSE
CH = 128
NCH = IDX_W // CH

TB = 512


def _sc_gather_body(idx_hbm, offs_hbm, tab_hbm, out_hbm,
                    idx_v, offs_v, rows_v, sem0, sem1):
    wid = lax.axis_index("s") * NC + lax.axis_index("c")
    sems = (sem0, sem1)

    pltpu.sync_copy(idx_hbm.at[pl.ds(wid * NCH, NCH)], idx_v)
    pltpu.sync_copy(offs_hbm, offs_v)

    def add_row(c, carry):
        for j in range(CH // 16):
            sl = pl.ds(j * 16, 16)
            idx_v[c, sl] = idx_v[c, sl] + offs_v[c, sl]
        return carry
    lax.fori_loop(0, NCH, add_row, 0)

    out_base = wid * IDX_W

    def start(c, slot):
        pltpu.async_copy(tab_hbm.at[idx_v.at[c]], rows_v.at[slot], sems[slot])

    def wait(c, slot):
        pltpu.make_async_copy(tab_hbm.at[idx_v.at[c]], rows_v.at[slot],
                              sems[slot]).wait()

    start(0, 0)
    start(1, 1)

    def pair(i, carry):
        c0 = i * 2
        for b in range(2):
            c = c0 + b
            wait(c, b)
            pltpu.sync_copy(rows_v.at[b],
                            out_hbm.at[pl.ds(out_base + c * CH, CH)])

            @pl.when(c + 2 < NCH)
            def _():
                start(c + 2, b)
        return carry
    lax.fori_loop(0, NCH // 2, pair, 0)


def _sc_gather(idx2d, offs, tab_flat):
    mesh = plsc.VectorSubcoreMesh(core_axis_name="c", subcore_axis_name="s",
                                  num_cores=NC, num_subcores=NSUB)
    f = pl.kernel(
        _sc_gather_body,
        out_type=jax.ShapeDtypeStruct((B * N_SPARSE, EDIM), jnp.float32),
        mesh=mesh,
        compiler_params=pltpu.CompilerParams(use_tc_tiling_on_sc=False),
        scratch_types=[
            pltpu.VMEM((NCH, CH), jnp.int32),
            pltpu.VMEM((NCH, CH), jnp.int32),
            pltpu.VMEM((2, CH, EDIM), jnp.float32),
            pltpu.SemaphoreType.DMA,
            pltpu.SemaphoreType.DMA,
        ],
    )
    return f(idx2d, offs, tab_flat)


def _mlp_body(emb_ref, x_ref, wlin_ref, blin_ref, w1e_ref, w1d_ref, b1_ref,
              w2_ref, b2_ref, wout_ref, o_ref):
    hi = jax.lax.Precision.HIGHEST
    x = x_ref[...]
    emb = emb_ref[...]
    xd = x[:, N_SPARSE:]

    wide = jnp.maximum(
        jnp.dot(x, wlin_ref[...], precision=hi,
                preferred_element_type=jnp.float32) + blin_ref[...], 0.0)

    h1 = jnp.dot(emb, w1e_ref[...], precision=hi,
                 preferred_element_type=jnp.float32)
    h1 = h1 + jnp.dot(xd, w1d_ref[...], precision=hi,
                      preferred_element_type=jnp.float32)
    h1 = jnp.maximum(h1 + b1_ref[...], 0.0)

    h2 = jnp.maximum(
        jnp.dot(h1, w2_ref[...], precision=hi,
                preferred_element_type=jnp.float32) + b2_ref[...], 0.0)

    z = wide + jnp.dot(h2, wout_ref[...], precision=hi,
                       preferred_element_type=jnp.float32)
    o_ref[...] = 1.0 / (1.0 + jnp.exp(-z))


def _mlp(emb, X, wlinT, blin, w1eT, w1dT, b1, w2T, b2, woutT):
    full = lambda shape: pl.BlockSpec(shape, lambda i: (0, 0))
    return pl.pallas_call(
        _mlp_body,
        grid=(B // TB,),
        in_specs=[
            pl.BlockSpec((TB, SPARSE_DIM), lambda i: (i, 0)),
            pl.BlockSpec((TB, N_FEAT), lambda i: (i, 0)),
            full((N_FEAT, 1)),
            full((1, 1)),
            full((SPARSE_DIM, HID1)),
            full((N_DENSE, HID1)),
            full((1, HID1)),
            full((HID1, HID2)),
            full((1, HID2)),
            full((HID2, 1)),
        ],
        out_specs=pl.BlockSpec((TB, 1), lambda i: (i, 0)),
        out_shape=jax.ShapeDtypeStruct((B, 1), jnp.float32),
    )(emb, X, wlinT, blin, w1eT, w1dT, b1, w2T, b2, woutT)


def kernel(X, tables, W_lin, b_lin, W1, b1, W2, b2, W_out):
    idx2d = X[:, :N_SPARSE].astype(jnp.int32).reshape(NW * NCH, CH)
    offs = jnp.tile(jnp.arange(N_SPARSE, dtype=jnp.int32) * VOCAB,
                    ROWS_W).reshape(NCH, CH)
    tab_flat = tables.reshape(N_SPARSE * VOCAB, EDIM)

    emb = _sc_gather(idx2d, offs, tab_flat).reshape(B, SPARSE_DIM)

    y = _mlp(emb, X,
             W_lin.T, b_lin.reshape(1, 1),
             W1[:, :SPARSE_DIM].T, W1[:, SPARSE_DIM:].T, b1.reshape(1, HID1),
             W2.T, b2.reshape(1, HID2),
             W_out.T)
    return y
